# speedup vs baseline: 4.1329x; 4.1329x over previous
import functools

import jax
import jax.numpy as jnp
from jax import lax
from jax.experimental import pallas as pl
from jax.experimental.pallas import tpu as pltpu
from jax.experimental.pallas import tpu_sc as plsc

NUM_TOKENS = 8192
DIM = 256
B_TOTAL = 64 * 1024
NUM_CORES = 2
NUM_SUBCORES = 16
NW = NUM_CORES * NUM_SUBCORES
BPW = B_TOTAL // NW
CHUNK = 128
NCHUNK = BPW // CHUNK

_mesh = plsc.VectorSubcoreMesh(core_axis_name="c", subcore_axis_name="s")


@functools.partial(
    pl.kernel,
    mesh=_mesh,
    out_type=jax.ShapeDtypeStruct((B_TOTAL, DIM), jnp.float32),
    scratch_types=[
        pltpu.VMEM((NCHUNK, CHUNK), jnp.int32),
        pltpu.VMEM((2, CHUNK, DIM), jnp.float32),
        pltpu.SemaphoreType.DMA,
        pltpu.SemaphoreType.DMA,
    ],
)
def _embed_lookup(idx_hbm, table_hbm, out_hbm, idx_v, rows_v, gsem, osem):
    wid = lax.axis_index("s") * NUM_CORES + lax.axis_index("c")
    base = wid * BPW

    pltpu.sync_copy(idx_hbm.at[wid], idx_v)

    gcp = [None, None]
    ocp = [None, None]
    gcp[0] = pltpu.async_copy(table_hbm.at[idx_v.at[0]], rows_v.at[0], gsem)
    for c in range(NCHUNK):
        buf = c & 1
        nbuf = buf ^ 1
        if c + 1 < NCHUNK:
            if ocp[nbuf] is not None:
                ocp[nbuf].wait()
            gcp[nbuf] = pltpu.async_copy(
                table_hbm.at[idx_v.at[c + 1]], rows_v.at[nbuf], gsem
            )
        gcp[buf].wait()
        ocp[buf] = pltpu.async_copy(
            rows_v.at[buf], out_hbm.at[pl.ds(base + c * CHUNK, CHUNK)], osem
        )
    ocp[0].wait()
    ocp[1].wait()


def kernel(embed_id, weight):
    idx = embed_id.reshape(NW, NCHUNK, CHUNK)
    out = _embed_lookup(idx, weight)
    return out.reshape(embed_id.shape[0], embed_id.shape[1], DIM)
